# Initial kernel scaffold; baseline (speedup 1.0000x reference)
import jax
import jax.numpy as jnp
from jax.experimental import pallas as pl


def kernel(xyzp, features, gamma, beta, W1, b1, W2, Pe_W1, Pe_b1, Pe_W2, Pe_b2, A_W1, A_b1, A_W2, A_b2, Ag_W, Ag_b, M_W1, M_b1, M_W2, M_b2):
    raise NotImplementedError("write your pallas kernel here")


# speedup vs baseline: 1.5086x; 1.5086x over previous
import functools
import math

import jax
import jax.numpy as jnp
from jax import lax
from jax.experimental import pallas as pl
from jax.experimental.pallas import tpu as pltpu
from jax.experimental.pallas import tpu_sc as plsc

B, N = 2, 2048
H, W = 512, 512
KNN = 16
FEAT, HID = 128, 128
P = B * N
BLK = 256
QBLK = 64


def _sc_gather(table, idx):
    V, D = table.shape
    Bt = idx.shape[0]
    NW = 32
    b_per_w = Bt // NW
    CH = 128
    n_ch = b_per_w // CH
    mesh = plsc.VectorSubcoreMesh(core_axis_name="c", subcore_axis_name="s")

    @functools.partial(
        pl.kernel,
        mesh=mesh,
        out_type=jax.ShapeDtypeStruct((Bt, D), jnp.float32),
        scratch_types=[
            pltpu.VMEM((CH,), jnp.int32),
            pltpu.VMEM((CH, D), jnp.float32),
            pltpu.SemaphoreType.DMA,
        ],
    )
    def k(table_hbm, idx_hbm, out_hbm, idx_v, rows_v, sem):
        wid = lax.axis_index("s") * 2 + lax.axis_index("c")
        base = wid * b_per_w
        for c in range(n_ch):
            off = base + c * CH
            pltpu.sync_copy(idx_hbm.at[pl.ds(off, CH)], idx_v)
            pltpu.async_copy(table_hbm.at[idx_v], rows_v, sem).wait()
            pltpu.sync_copy(rows_v, out_hbm.at[pl.ds(off, CH)])

    return k(table, idx)


def _k1_body(xz_ref, f_ref, g_ref, b_ref, out_ref):
    xz = xz_ref[...]
    f = f_ref[...]
    p = (xz[:, 3:4] - 0.5) / 0.5
    pos = jnp.clip(p, 0.0, 1.0)
    neg = -jnp.clip(p, -1.0, 0.0)
    s1 = jnp.sum(f, axis=1, keepdims=True) + pos + neg
    mean = s1 / 130.0
    s2 = jnp.sum(f * f, axis=1, keepdims=True) + pos * pos + neg * neg
    var = s2 / 130.0 - mean * mean
    inv = lax.rsqrt(var + 1e-5)
    row = jnp.concatenate(
        [pos, neg, f, jnp.zeros((BLK, 126), jnp.float32)], axis=1)
    mean_b = jnp.concatenate(
        [mean, mean, jnp.broadcast_to(mean, (BLK, 128)),
         jnp.zeros((BLK, 126), jnp.float32)], axis=1)
    out_ref[...] = (row - mean_b) * inv * g_ref[...] + b_ref[...]


def _run_k1(xyzp4, feat2, g_row, b_row):
    return pl.pallas_call(
        _k1_body,
        grid=(P // BLK,),
        in_specs=[
            pl.BlockSpec((BLK, 4), lambda i: (i, 0)),
            pl.BlockSpec((BLK, 128), lambda i: (i, 0)),
            pl.BlockSpec((1, 256), lambda i: (0, 0)),
            pl.BlockSpec((1, 256), lambda i: (0, 0)),
        ],
        out_specs=pl.BlockSpec((BLK, 256), lambda i: (i, 0)),
        out_shape=jax.ShapeDtypeStruct((P, 256), jnp.float32),
    )(xyzp4, feat2, g_row, b_row)


def _k2_body(xz_ref, xyT_ref, out_ref):
    b = pl.program_id(0)
    xz = xz_ref[...]
    xyT = xyT_ref[...][0]
    xyb = jnp.concatenate(
        [xz[:, 0:2], jnp.zeros((BLK, 6), jnp.float32)], axis=1)
    sq_all = jnp.sum(xyT * xyT, axis=0, keepdims=True)
    sqb = jnp.sum(xyb * xyb, axis=1, keepdims=True)
    mm = jnp.dot(xyb, xyT, preferred_element_type=jnp.float32)
    d2 = sqb + sq_all - 2.0 * mm
    iota = lax.broadcasted_iota(jnp.int32, (BLK, N), 1)
    cols = []
    for _ in range(KNN):
        m = jnp.min(d2, axis=1, keepdims=True)
        am = jnp.min(jnp.where(d2 == m, iota, N), axis=1, keepdims=True)
        cols.append(am)
        d2 = jnp.where(iota == am, jnp.float32(jnp.inf), d2)
    idx = jnp.concatenate(cols, axis=1)
    out_ref[...] = idx + b * N


def _run_k2(xyzp4, xyT):
    return pl.pallas_call(
        _k2_body,
        grid=(B, N // BLK),
        in_specs=[
            pl.BlockSpec((BLK, 4), lambda b, i: (b * (N // BLK) + i, 0)),
            pl.BlockSpec((1, 8, N), lambda b, i: (b, 0, 0)),
        ],
        out_specs=pl.BlockSpec((BLK, KNN), lambda b, i: (b * (N // BLK) + i, 0)),
        out_shape=jax.ShapeDtypeStruct((P, KNN), jnp.int32),
    )(xyzp4, xyT)


def _k3_body(xz_ref, xzT_ref, out_ref):
    pid = pl.program_id(0)
    b_s = pid // ((N // BLK))
    xz = xz_ref[...]
    yi = jnp.clip(jnp.round(xz[:, 1:2] * H).astype(jnp.int32), 0, H - 1)
    xi = jnp.clip(jnp.round(xz[:, 0:2][:, 0:1] * W).astype(jnp.int32), 0, W - 1)
    key_col = b_s * (H * W) + yi * W + xi
    xzT = xzT_ref[...]
    y_row = jnp.clip(jnp.round(xzT[1:2, :] * H).astype(jnp.int32), 0, H - 1)
    x_row = jnp.clip(jnp.round(xzT[0:1, :] * W).astype(jnp.int32), 0, W - 1)
    b_row = lax.broadcasted_iota(jnp.int32, (1, P), 1) // N
    key_row = b_row * (H * W) + y_row * W + x_row
    iota = lax.broadcasted_iota(jnp.int32, (BLK, P), 1)
    cols = []
    for dy in (-1, 0, 1):
        for dx in (-1, 0, 1):
            valid = ((yi + dy >= 0) & (yi + dy < H) &
                     (xi + dx >= 0) & (xi + dx < W))
            nkey = jnp.where(valid, key_col + dy * W + dx, -1)
            match = nkey == key_row
            cols.append(jnp.min(jnp.where(match, iota, P), axis=1,
                                keepdims=True))
    cols.append(jnp.full((BLK, 7), P, jnp.int32))
    out_ref[...] = jnp.concatenate(cols, axis=1)


def _run_k3(xyzp4, xyzpT):
    return pl.pallas_call(
        _k3_body,
        grid=(P // BLK,),
        in_specs=[
            pl.BlockSpec((BLK, 4), lambda i: (i, 0)),
            pl.BlockSpec((4, P), lambda i: (0, 0)),
        ],
        out_specs=pl.BlockSpec((BLK, 16), lambda i: (i, 0)),
        out_shape=jax.ShapeDtypeStruct((P, 16), jnp.int32),
    )(xyzp4, xyzpT)


def _k4_body(g_ref, w_ref, b_ref, out_ref):
    g = g_ref[...]
    w = w_ref[...]
    acc = jnp.broadcast_to(b_ref[...], (BLK, HID))
    for t in range(9):
        acc = acc + jnp.dot(g[:, t, :], w[t],
                            preferred_element_type=jnp.float32)
    out_ref[...] = acc


def _run_k4(g1, W1p, b1row):
    return pl.pallas_call(
        _k4_body,
        grid=(P // BLK,),
        in_specs=[
            pl.BlockSpec((BLK, 9, 256), lambda i: (i, 0, 0)),
            pl.BlockSpec((9, 256, 128), lambda i: (0, 0, 0)),
            pl.BlockSpec((1, 128), lambda i: (0, 0)),
        ],
        out_specs=pl.BlockSpec((BLK, HID), lambda i: (i, 0)),
        out_shape=jax.ShapeDtypeStruct((P, HID), jnp.float32),
    )(g1, W1p, b1row)


def _k5_body(g_ref, w_ref, xz_ref, q_ref, tab_ref):
    g = g_ref[...]
    w = w_ref[...]
    acc = jnp.zeros((BLK, 3 * HID), jnp.float32)
    for t in range(9):
        acc = acc + jnp.dot(g[:, t, :], w[t],
                            preferred_element_type=jnp.float32)
    q_ref[...] = acc[:, 0:HID]
    xz = xz_ref[...]
    xyp = jnp.concatenate(
        [xz[:, 0:2], xz[:, 3:4], jnp.zeros((BLK, 125), jnp.float32)], axis=1)
    tab_ref[...] = jnp.concatenate(
        [acc[:, HID:2 * HID], acc[:, 2 * HID:3 * HID], xyp], axis=1)


def _run_k5(g2, W2g, xyzp4):
    return pl.pallas_call(
        _k5_body,
        grid=(P // BLK,),
        in_specs=[
            pl.BlockSpec((BLK, 9, 128), lambda i: (i, 0, 0)),
            pl.BlockSpec((9, 128, 384), lambda i: (0, 0, 0)),
            pl.BlockSpec((BLK, 4), lambda i: (i, 0)),
        ],
        out_specs=[
            pl.BlockSpec((BLK, HID), lambda i: (i, 0)),
            pl.BlockSpec((BLK, 384), lambda i: (i, 0)),
        ],
        out_shape=[
            jax.ShapeDtypeStruct((P, HID), jnp.float32),
            jax.ShapeDtypeStruct((P, 384), jnp.float32),
        ],
    )(g2, W2g, xyzp4)


def _k6_body(q_ref, g_ref, xz_ref, f_ref,
             pew1_ref, peb1_ref, pew2_ref, peb2_ref,
             aw1_ref, ab1_ref, aw2_ref, ab2_ref,
             agw_ref, agb_ref, mw1_ref, mb1_ref, mw2_ref, mb2_ref,
             out_ref):
    R = QBLK * KNN
    g = g_ref[...]
    xz = xz_ref[...]
    xyp_q = jnp.concatenate(
        [xz[:, 0:2], xz[:, 3:4], jnp.zeros((QBLK, 29), jnp.float32)], axis=1)
    xq = jnp.reshape(
        jnp.broadcast_to(jnp.reshape(xyp_q, (QBLK, 1, 32)), (QBLK, KNN, 32)),
        (R, 32))
    rel = xq - g[:, 256:288]
    pe = jnp.maximum(
        jnp.dot(rel, pew1_ref[...], preferred_element_type=jnp.float32)
        + peb1_ref[...], 0.0)
    pe = jnp.dot(pe, pew2_ref[...],
                 preferred_element_type=jnp.float32) + peb2_ref[...]
    q = q_ref[...]
    qe = jnp.reshape(
        jnp.broadcast_to(jnp.reshape(q, (QBLK, 1, HID)), (QBLK, KNN, HID)),
        (R, HID))
    att = qe - g[:, 0:HID] + pe
    att = jnp.maximum(
        jnp.dot(att, aw1_ref[...], preferred_element_type=jnp.float32)
        + ab1_ref[...], 0.0)
    att = jnp.dot(att, aw2_ref[...],
                  preferred_element_type=jnp.float32) + ab2_ref[...]
    att = att / math.sqrt(HID)
    a3 = jnp.reshape(att, (QBLK, KNN, HID))
    mx = jnp.max(a3, axis=1, keepdims=True)
    e = jnp.exp(a3 - mx)
    sm = e / jnp.sum(e, axis=1, keepdims=True)
    v3 = jnp.reshape(g[:, HID:2 * HID] + pe, (QBLK, KNN, HID))
    res = jnp.sum(sm * v3, axis=1)
    f = f_ref[...]
    res = jnp.dot(res, agw_ref[...],
                  preferred_element_type=jnp.float32) + agb_ref[...] + f
    h = jnp.dot(res, mw1_ref[...],
                preferred_element_type=jnp.float32) + mb1_ref[...]
    h = 0.5 * h * (1.0 + lax.erf(h / math.sqrt(2.0)))
    out_ref[...] = jnp.dot(h, mw2_ref[...],
                           preferred_element_type=jnp.float32) + mb2_ref[...] + f


def _run_k6(q, ga, xyzp4, feat2, weights):
    full = lambda r, c: pl.BlockSpec((r, c), lambda i: (0, 0))
    return pl.pallas_call(
        _k6_body,
        grid=(P // QBLK,),
        in_specs=[
            pl.BlockSpec((QBLK, HID), lambda i: (i, 0)),
            pl.BlockSpec((QBLK * KNN, 384), lambda i: (i, 0)),
            pl.BlockSpec((QBLK, 4), lambda i: (i, 0)),
            pl.BlockSpec((QBLK, 128), lambda i: (i, 0)),
            full(32, 128), full(1, 128), full(128, 128), full(1, 128),
            full(128, 128), full(1, 128), full(128, 128), full(1, 128),
            full(128, 128), full(1, 128), full(128, 128), full(1, 128),
            full(128, 128), full(1, 128),
        ],
        out_specs=pl.BlockSpec((QBLK, FEAT), lambda i: (i, 0)),
        out_shape=jax.ShapeDtypeStruct((P, FEAT), jnp.float32),
    )(q, ga, xyzp4, feat2, *weights)


def kernel(xyzp, features, gamma, beta, W1, b1, W2, Pe_W1, Pe_b1, Pe_W2,
           Pe_b2, A_W1, A_b1, A_W2, A_b2, Ag_W, Ag_b, M_W1, M_b1, M_W2, M_b2):
    xyzp4 = xyzp.reshape(P, 4)
    feat2 = features.reshape(P, FEAT)
    g_row = jnp.pad(gamma, (0, 126)).reshape(1, 256)
    b_row = jnp.pad(beta, (0, 126)).reshape(1, 256)
    xyT = jnp.pad(jnp.transpose(xyzp[..., 0:2], (0, 2, 1)),
                  ((0, 0), (0, 6), (0, 0)))
    xyzpT = xyzp4.T
    W1p = jnp.pad(W1.reshape(9, FEAT + 2, HID), ((0, 0), (0, 126), (0, 0)))
    perm = jnp.concatenate(
        [3 * jnp.arange(HID, dtype=jnp.int32) + c for c in range(3)])
    W2g = W2.reshape(9, HID, 3 * HID)[:, :, perm]
    b1row = b1.reshape(1, HID)
    row = lambda x: x.reshape(1, 128)
    pew1p = jnp.pad(Pe_W1, ((0, 29), (0, 0)))

    sparse = _run_k1(xyzp4, feat2, g_row, b_row)
    idx = _run_k2(xyzp4, xyT)
    nidx = _run_k3(xyzp4, xyzpT)
    conv_idx = nidx[:, :9].reshape(-1)
    att_idx = idx.reshape(-1)

    table1 = jnp.concatenate([sparse, jnp.zeros((8, 256), jnp.float32)])
    g1 = _sc_gather(table1, conv_idx).reshape(P, 9, 256)
    h1 = _run_k4(g1, W1p, b1row)
    table2 = jnp.concatenate([h1, jnp.zeros((8, HID), jnp.float32)])
    g2 = _sc_gather(table2, conv_idx).reshape(P, 9, HID)
    q, att_table = _run_k5(g2, W2g, xyzp4)
    ga = _sc_gather(att_table, att_idx)

    weights = (pew1p, row(Pe_b1), Pe_W2, row(Pe_b2),
               A_W1, row(A_b1), A_W2, row(A_b2),
               Ag_W, row(Ag_b), M_W1, row(M_b1), M_W2, row(M_b2))
    out = _run_k6(q, ga, xyzp4, feat2, weights)
    return out.reshape(B, N, FEAT)


# speedup vs baseline: 6.5778x; 4.3601x over previous
import functools
import math

import jax
import jax.numpy as jnp
from jax import lax
from jax.experimental import pallas as pl
from jax.experimental.pallas import tpu as pltpu
from jax.experimental.pallas import tpu_sc as plsc

B, N = 2, 2048
H, W = 512, 512
KNN = 16
FEAT, HID = 128, 128
P = B * N
BLK = 256
QBLK = 64


def _sc_gather(table, idx):
    V, D = table.shape
    Bt = idx.shape[0]
    NW = 32
    b_per_w = Bt // NW
    CH = 128
    n_ch = b_per_w // CH
    mesh = plsc.VectorSubcoreMesh(core_axis_name="c", subcore_axis_name="s")

    @functools.partial(
        pl.kernel,
        mesh=mesh,
        out_type=jax.ShapeDtypeStruct((Bt, D), jnp.float32),
        scratch_types=[
            pltpu.VMEM((CH,), jnp.int32),
            pltpu.VMEM((CH, D), jnp.float32),
            pltpu.SemaphoreType.DMA,
        ],
    )
    def k(table_hbm, idx_hbm, out_hbm, idx_v, rows_v, sem):
        wid = lax.axis_index("s") * 2 + lax.axis_index("c")
        base = wid * b_per_w
        for c in range(n_ch):
            off = base + c * CH
            pltpu.sync_copy(idx_hbm.at[pl.ds(off, CH)], idx_v)
            pltpu.async_copy(table_hbm.at[idx_v], rows_v, sem).wait()
            pltpu.sync_copy(rows_v, out_hbm.at[pl.ds(off, CH)])

    return k(table, idx)


def _k1_body(xz_ref, f_ref, g_ref, b_ref, out_ref):
    xz = xz_ref[...]
    f = f_ref[...]
    p = (xz[:, 3:4] - 0.5) / 0.5
    pos = jnp.clip(p, 0.0, 1.0)
    neg = -jnp.clip(p, -1.0, 0.0)
    s1 = jnp.sum(f, axis=1, keepdims=True) + pos + neg
    mean = s1 / 130.0
    s2 = jnp.sum(f * f, axis=1, keepdims=True) + pos * pos + neg * neg
    var = s2 / 130.0 - mean * mean
    inv = lax.rsqrt(var + 1e-5)
    row = jnp.concatenate(
        [pos, neg, f, jnp.zeros((BLK, 126), jnp.float32)], axis=1)
    mean_b = jnp.concatenate(
        [mean, mean, jnp.broadcast_to(mean, (BLK, 128)),
         jnp.zeros((BLK, 126), jnp.float32)], axis=1)
    out_ref[...] = (row - mean_b) * inv * g_ref[...] + b_ref[...]


def _run_k1(xyzp4, feat2, g_row, b_row):
    return pl.pallas_call(
        _k1_body,
        grid=(P // BLK,),
        in_specs=[
            pl.BlockSpec((BLK, 4), lambda i: (i, 0)),
            pl.BlockSpec((BLK, 128), lambda i: (i, 0)),
            pl.BlockSpec((1, 256), lambda i: (0, 0)),
            pl.BlockSpec((1, 256), lambda i: (0, 0)),
        ],
        out_specs=pl.BlockSpec((BLK, 256), lambda i: (i, 0)),
        out_shape=jax.ShapeDtypeStruct((P, 256), jnp.float32),
    )(xyzp4, feat2, g_row, b_row)


def _k2_body(xz_ref, xyT_ref, out_ref):
    b = pl.program_id(0)
    xz = xz_ref[...]
    xyT = xyT_ref[...][0]
    xyb = jnp.concatenate(
        [xz[:, 0:2], jnp.zeros((BLK, 6), jnp.float32)], axis=1)
    sq_all = jnp.sum(xyT * xyT, axis=0, keepdims=True)
    sqb = jnp.sum(xyb * xyb, axis=1, keepdims=True)
    mm = jnp.dot(xyb, xyT, preferred_element_type=jnp.float32)
    d2 = sqb + sq_all - 2.0 * mm
    iota = lax.broadcasted_iota(jnp.int32, (BLK, N), 1)
    cols = []
    for _ in range(KNN):
        m = jnp.min(d2, axis=1, keepdims=True)
        am = jnp.min(jnp.where(d2 == m, iota, N), axis=1, keepdims=True)
        cols.append(am)
        d2 = jnp.where(iota == am, jnp.float32(jnp.inf), d2)
    idx = jnp.concatenate(cols, axis=1)
    out_ref[...] = idx + b * N


def _run_k2(xyzp4, xyT):
    return pl.pallas_call(
        _k2_body,
        grid=(B, N // BLK),
        in_specs=[
            pl.BlockSpec((BLK, 4), lambda b, i: (b * (N // BLK) + i, 0)),
            pl.BlockSpec((1, 8, N), lambda b, i: (b, 0, 0)),
        ],
        out_specs=pl.BlockSpec((BLK, KNN), lambda b, i: (b * (N // BLK) + i, 0)),
        out_shape=jax.ShapeDtypeStruct((P, KNN), jnp.int32),
    )(xyzp4, xyT)


def _k3_body(xz_ref, xzT_ref, out_ref, msk_ref):
    pid = pl.program_id(0)
    b_s = pid // ((N // BLK))
    xz = xz_ref[...]
    yi = jnp.clip(jnp.round(xz[:, 1:2] * H).astype(jnp.int32), 0, H - 1)
    xi = jnp.clip(jnp.round(xz[:, 0:2][:, 0:1] * W).astype(jnp.int32), 0, W - 1)
    key_col = b_s * (H * W) + yi * W + xi
    xzT = xzT_ref[...]
    y_row = jnp.clip(jnp.round(xzT[1:2, :] * H).astype(jnp.int32), 0, H - 1)
    x_row = jnp.clip(jnp.round(xzT[0:1, :] * W).astype(jnp.int32), 0, W - 1)
    b_row = lax.broadcasted_iota(jnp.int32, (1, P), 1) // N
    key_row = b_row * (H * W) + y_row * W + x_row
    iota = lax.broadcasted_iota(jnp.int32, (BLK, P), 1)
    self_col = lax.broadcasted_iota(jnp.int32, (BLK, 1), 0) + pid * BLK
    cols = []
    mcols = []
    for dy in (-1, 0, 1):
        for dx in (-1, 0, 1):
            valid = ((yi + dy >= 0) & (yi + dy < H) &
                     (xi + dx >= 0) & (xi + dx < W))
            nkey = jnp.where(valid, key_col + dy * W + dx, -1)
            match = nkey == key_row
            hit = jnp.min(jnp.where(match, iota, P), axis=1, keepdims=True)
            mcols.append(jnp.where(hit == P, 0.0, 1.0))
            cols.append(jnp.where(hit == P, self_col, hit))
    cols.append(jnp.full((BLK, 7), 0, jnp.int32))
    mcols.append(jnp.zeros((BLK, 7), jnp.float32))
    out_ref[...] = jnp.concatenate(cols, axis=1)
    msk_ref[...] = jnp.concatenate(mcols, axis=1)


def _run_k3(xyzp4, xyzpT):
    return pl.pallas_call(
        _k3_body,
        grid=(P // BLK,),
        in_specs=[
            pl.BlockSpec((BLK, 4), lambda i: (i, 0)),
            pl.BlockSpec((4, P), lambda i: (0, 0)),
        ],
        out_specs=[
            pl.BlockSpec((BLK, 16), lambda i: (i, 0)),
            pl.BlockSpec((BLK, 16), lambda i: (i, 0)),
        ],
        out_shape=[
            jax.ShapeDtypeStruct((P, 16), jnp.int32),
            jax.ShapeDtypeStruct((P, 16), jnp.float32),
        ],
    )(xyzp4, xyzpT)


def _k4_body(g_ref, w_ref, b_ref, m_ref, out_ref):
    g = g_ref[...]
    w = w_ref[...]
    m = m_ref[...]
    acc = jnp.broadcast_to(b_ref[...], (BLK, HID))
    for t in range(9):
        acc = acc + jnp.dot(g[:, t, :] * m[:, t:t + 1], w[t],
                            preferred_element_type=jnp.float32)
    out_ref[...] = acc


def _run_k4(g1, W1p, b1row, mask):
    return pl.pallas_call(
        _k4_body,
        grid=(P // BLK,),
        in_specs=[
            pl.BlockSpec((BLK, 9, 256), lambda i: (i, 0, 0)),
            pl.BlockSpec((9, 256, 128), lambda i: (0, 0, 0)),
            pl.BlockSpec((1, 128), lambda i: (0, 0)),
            pl.BlockSpec((BLK, 16), lambda i: (i, 0)),
        ],
        out_specs=pl.BlockSpec((BLK, HID), lambda i: (i, 0)),
        out_shape=jax.ShapeDtypeStruct((P, HID), jnp.float32),
    )(g1, W1p, b1row, mask)


def _k5_body(g_ref, w_ref, xz_ref, m_ref, q_ref, tab_ref):
    g = g_ref[...]
    w = w_ref[...]
    m = m_ref[...]
    acc = jnp.zeros((BLK, 3 * HID), jnp.float32)
    for t in range(9):
        acc = acc + jnp.dot(g[:, t, :] * m[:, t:t + 1], w[t],
                            preferred_element_type=jnp.float32)
    q_ref[...] = acc[:, 0:HID]
    xz = xz_ref[...]
    xyp = jnp.concatenate(
        [xz[:, 0:2], xz[:, 3:4], jnp.zeros((BLK, 125), jnp.float32)], axis=1)
    tab_ref[...] = jnp.concatenate(
        [acc[:, HID:2 * HID], acc[:, 2 * HID:3 * HID], xyp], axis=1)


def _run_k5(g2, W2g, xyzp4, mask):
    return pl.pallas_call(
        _k5_body,
        grid=(P // BLK,),
        in_specs=[
            pl.BlockSpec((BLK, 9, 128), lambda i: (i, 0, 0)),
            pl.BlockSpec((9, 128, 384), lambda i: (0, 0, 0)),
            pl.BlockSpec((BLK, 4), lambda i: (i, 0)),
            pl.BlockSpec((BLK, 16), lambda i: (i, 0)),
        ],
        out_specs=[
            pl.BlockSpec((BLK, HID), lambda i: (i, 0)),
            pl.BlockSpec((BLK, 384), lambda i: (i, 0)),
        ],
        out_shape=[
            jax.ShapeDtypeStruct((P, HID), jnp.float32),
            jax.ShapeDtypeStruct((P, 384), jnp.float32),
        ],
    )(g2, W2g, xyzp4, mask)


def _k6_body(q_ref, g_ref, xz_ref, f_ref,
             pew1_ref, peb1_ref, pew2_ref, peb2_ref,
             aw1_ref, ab1_ref, aw2_ref, ab2_ref,
             agw_ref, agb_ref, mw1_ref, mb1_ref, mw2_ref, mb2_ref,
             out_ref):
    R = QBLK * KNN
    g = g_ref[...]
    xz = xz_ref[...]
    xyp_q = jnp.concatenate(
        [xz[:, 0:2], xz[:, 3:4], jnp.zeros((QBLK, 29), jnp.float32)], axis=1)
    xq = jnp.reshape(
        jnp.broadcast_to(jnp.reshape(xyp_q, (QBLK, 1, 32)), (QBLK, KNN, 32)),
        (R, 32))
    rel = xq - g[:, 256:288]
    pe = jnp.maximum(
        jnp.dot(rel, pew1_ref[...], preferred_element_type=jnp.float32)
        + peb1_ref[...], 0.0)
    pe = jnp.dot(pe, pew2_ref[...],
                 preferred_element_type=jnp.float32) + peb2_ref[...]
    q = q_ref[...]
    qe = jnp.reshape(
        jnp.broadcast_to(jnp.reshape(q, (QBLK, 1, HID)), (QBLK, KNN, HID)),
        (R, HID))
    att = qe - g[:, 0:HID] + pe
    att = jnp.maximum(
        jnp.dot(att, aw1_ref[...], preferred_element_type=jnp.float32)
        + ab1_ref[...], 0.0)
    att = jnp.dot(att, aw2_ref[...],
                  preferred_element_type=jnp.float32) + ab2_ref[...]
    att = att / math.sqrt(HID)
    a3 = jnp.reshape(att, (QBLK, KNN, HID))
    mx = jnp.max(a3, axis=1, keepdims=True)
    e = jnp.exp(a3 - mx)
    sm = e / jnp.sum(e, axis=1, keepdims=True)
    v3 = jnp.reshape(g[:, HID:2 * HID] + pe, (QBLK, KNN, HID))
    res = jnp.sum(sm * v3, axis=1)
    f = f_ref[...]
    res = jnp.dot(res, agw_ref[...],
                  preferred_element_type=jnp.float32) + agb_ref[...] + f
    h = jnp.dot(res, mw1_ref[...],
                preferred_element_type=jnp.float32) + mb1_ref[...]
    h = 0.5 * h * (1.0 + lax.erf(h / math.sqrt(2.0)))
    out_ref[...] = jnp.dot(h, mw2_ref[...],
                           preferred_element_type=jnp.float32) + mb2_ref[...] + f


def _run_k6(q, ga, xyzp4, feat2, weights):
    full = lambda r, c: pl.BlockSpec((r, c), lambda i: (0, 0))
    return pl.pallas_call(
        _k6_body,
        grid=(P // QBLK,),
        in_specs=[
            pl.BlockSpec((QBLK, HID), lambda i: (i, 0)),
            pl.BlockSpec((QBLK * KNN, 384), lambda i: (i, 0)),
            pl.BlockSpec((QBLK, 4), lambda i: (i, 0)),
            pl.BlockSpec((QBLK, 128), lambda i: (i, 0)),
            full(32, 128), full(1, 128), full(128, 128), full(1, 128),
            full(128, 128), full(1, 128), full(128, 128), full(1, 128),
            full(128, 128), full(1, 128), full(128, 128), full(1, 128),
            full(128, 128), full(1, 128),
        ],
        out_specs=pl.BlockSpec((QBLK, FEAT), lambda i: (i, 0)),
        out_shape=jax.ShapeDtypeStruct((P, FEAT), jnp.float32),
    )(q, ga, xyzp4, feat2, *weights)


def kernel(xyzp, features, gamma, beta, W1, b1, W2, Pe_W1, Pe_b1, Pe_W2,
           Pe_b2, A_W1, A_b1, A_W2, A_b2, Ag_W, Ag_b, M_W1, M_b1, M_W2, M_b2):
    xyzp4 = xyzp.reshape(P, 4)
    feat2 = features.reshape(P, FEAT)
    g_row = jnp.pad(gamma, (0, 126)).reshape(1, 256)
    b_row = jnp.pad(beta, (0, 126)).reshape(1, 256)
    xyT = jnp.pad(jnp.transpose(xyzp[..., 0:2], (0, 2, 1)),
                  ((0, 0), (0, 6), (0, 0)))
    xyzpT = xyzp4.T
    W1p = jnp.pad(W1.reshape(9, FEAT + 2, HID), ((0, 0), (0, 126), (0, 0)))
    perm = jnp.concatenate(
        [3 * jnp.arange(HID, dtype=jnp.int32) + c for c in range(3)])
    W2g = W2.reshape(9, HID, 3 * HID)[:, :, perm]
    b1row = b1.reshape(1, HID)
    row = lambda x: x.reshape(1, 128)
    pew1p = jnp.pad(Pe_W1, ((0, 29), (0, 0)))

    sparse = _run_k1(xyzp4, feat2, g_row, b_row)
    idx = _run_k2(xyzp4, xyT)
    nidx, mask = _run_k3(xyzp4, xyzpT)
    conv_idx = nidx[:, :9].reshape(-1)
    att_idx = idx.reshape(-1)

    g1 = _sc_gather(sparse, conv_idx).reshape(P, 9, 256)
    h1 = _run_k4(g1, W1p, b1row, mask)
    g2 = _sc_gather(h1, conv_idx).reshape(P, 9, HID)
    q, att_table = _run_k5(g2, W2g, xyzp4, mask)
    ga = _sc_gather(att_table, att_idx)

    weights = (pew1p, row(Pe_b1), Pe_W2, row(Pe_b2),
               A_W1, row(A_b1), A_W2, row(A_b2),
               Ag_W, row(Ag_b), M_W1, row(M_b1), M_W2, row(M_b2))
    out = _run_k6(q, ga, xyzp4, feat2, weights)
    return out.reshape(B, N, FEAT)


# speedup vs baseline: 6.8109x; 1.0354x over previous
import functools
import math

import jax
import jax.numpy as jnp
from jax import lax
from jax.experimental import pallas as pl
from jax.experimental.pallas import tpu as pltpu
from jax.experimental.pallas import tpu_sc as plsc

B, N = 2, 2048
H, W = 512, 512
KNN = 16
FEAT, HID = 128, 128
P = B * N
BLK = 256
QBLK = 64


def _sc_gather(table, idx):
    V, D = table.shape
    Bt = idx.shape[0]
    NW = 32
    b_per_w = Bt // NW
    CH = 128
    n_ch = b_per_w // CH
    mesh = plsc.VectorSubcoreMesh(core_axis_name="c", subcore_axis_name="s")

    @functools.partial(
        pl.kernel,
        mesh=mesh,
        out_type=jax.ShapeDtypeStruct((Bt, D), jnp.float32),
        scratch_types=[
            pltpu.VMEM((CH,), jnp.int32),
            pltpu.VMEM((CH, D), jnp.float32),
            pltpu.SemaphoreType.DMA,
        ],
    )
    def k(table_hbm, idx_hbm, out_hbm, idx_v, rows_v, sem):
        wid = lax.axis_index("s") * 2 + lax.axis_index("c")
        base = wid * b_per_w
        for c in range(n_ch):
            off = base + c * CH
            pltpu.sync_copy(idx_hbm.at[pl.ds(off, CH)], idx_v)
            pltpu.async_copy(table_hbm.at[idx_v], rows_v, sem).wait()
            pltpu.sync_copy(rows_v, out_hbm.at[pl.ds(off, CH)])

    return k(table, idx)


def _k123_body(xz_ref, f_ref, g_ref, b_ref, xyT_ref, xzT_ref,
               sp_ref, idx_ref, nidx_ref, msk_ref):
    pid = pl.program_id(0)
    b_s = pid // (N // BLK)
    xz = xz_ref[...]
    f = f_ref[...]
    p = (xz[:, 3:4] - 0.5) / 0.5
    pos = jnp.clip(p, 0.0, 1.0)
    neg = -jnp.clip(p, -1.0, 0.0)
    s1 = jnp.sum(f, axis=1, keepdims=True) + pos + neg
    mean = s1 / 130.0
    s2 = jnp.sum(f * f, axis=1, keepdims=True) + pos * pos + neg * neg
    var = s2 / 130.0 - mean * mean
    inv = lax.rsqrt(var + 1e-5)
    row = jnp.concatenate(
        [pos, neg, f, jnp.zeros((BLK, 126), jnp.float32)], axis=1)
    mean_b = jnp.concatenate(
        [mean, mean, jnp.broadcast_to(mean, (BLK, 128)),
         jnp.zeros((BLK, 126), jnp.float32)], axis=1)
    sp_ref[...] = (row - mean_b) * inv * g_ref[...] + b_ref[...]
    xyT = xyT_ref[...][0]
    xyb = jnp.concatenate(
        [xz[:, 0:2], jnp.zeros((BLK, 6), jnp.float32)], axis=1)
    sq_all = jnp.sum(xyT * xyT, axis=0, keepdims=True)
    sqb = jnp.sum(xyb * xyb, axis=1, keepdims=True)
    mm = jnp.dot(xyb, xyT, preferred_element_type=jnp.float32)
    d2 = sqb + sq_all - 2.0 * mm
    iota = lax.broadcasted_iota(jnp.int32, (BLK, N), 1)
    inf = jnp.float32(float("inf"))
    cols = []
    for _ in range(KNN):
        m = jnp.min(d2, axis=1, keepdims=True)
        hit = jnp.min(jnp.where(d2 == m, iota, N), axis=1, keepdims=True)
        cols.append(hit)
        d2 = jnp.where(iota == hit, inf, d2)
    idx = jnp.concatenate(cols, axis=1)
    idx_ref[...] = idx + b_s * N
    yi = jnp.clip(jnp.round(xz[:, 1:2] * H).astype(jnp.int32), 0, H - 1)
    xi = jnp.clip(jnp.round(xz[:, 0:1] * W).astype(jnp.int32), 0, W - 1)
    key_col = yi * W + xi
    xzT = xzT_ref[...]
    y_row = jnp.clip(jnp.round(xzT[1:2, :] * H).astype(jnp.int32), 0, H - 1)
    x_row = jnp.clip(jnp.round(xzT[0:1, :] * W).astype(jnp.int32), 0, W - 1)
    key_row = y_row * W + x_row
    delta = key_row - key_col
    self_col = (lax.broadcasted_iota(jnp.int32, (BLK, 1), 0)
                + (pid % (N // BLK)) * BLK)
    cols = []
    mcols = []
    for dy in (-1, 0, 1):
        for dx in (-1, 0, 1):
            valid = ((yi + dy >= 0) & (yi + dy < H) &
                     (xi + dx >= 0) & (xi + dx < W))
            cond = (delta == (dy * W + dx)) & valid
            hit = jnp.min(jnp.where(cond, iota, N), axis=1, keepdims=True)
            mcols.append(jnp.where(hit == N, 0.0, 1.0))
            cols.append(jnp.where(hit == N, self_col, hit))
    cols.append(jnp.full((BLK, 7), 0, jnp.int32))
    mcols.append(jnp.zeros((BLK, 7), jnp.float32))
    nidx_ref[...] = jnp.concatenate(cols, axis=1) + b_s * N
    msk_ref[...] = jnp.concatenate(mcols, axis=1)


def _run_k123(xyzp4, feat2, g_row, b_row, xyT, xyzpT):
    nb = N // BLK
    return pl.pallas_call(
        _k123_body,
        grid=(P // BLK,),
        in_specs=[
            pl.BlockSpec((BLK, 4), lambda i: (i, 0)),
            pl.BlockSpec((BLK, 128), lambda i: (i, 0)),
            pl.BlockSpec((1, 256), lambda i: (0, 0)),
            pl.BlockSpec((1, 256), lambda i: (0, 0)),
            pl.BlockSpec((1, 8, N), lambda i: (i // nb, 0, 0)),
            pl.BlockSpec((4, N), lambda i: (0, i // nb)),
        ],
        out_specs=[
            pl.BlockSpec((BLK, 256), lambda i: (i, 0)),
            pl.BlockSpec((BLK, KNN), lambda i: (i, 0)),
            pl.BlockSpec((BLK, 16), lambda i: (i, 0)),
            pl.BlockSpec((BLK, 16), lambda i: (i, 0)),
        ],
        out_shape=[
            jax.ShapeDtypeStruct((P, 256), jnp.float32),
            jax.ShapeDtypeStruct((P, KNN), jnp.int32),
            jax.ShapeDtypeStruct((P, 16), jnp.int32),
            jax.ShapeDtypeStruct((P, 16), jnp.float32),
        ],
    )(xyzp4, feat2, g_row, b_row, xyT, xyzpT)


def _k4_body(g_ref, w_ref, b_ref, m_ref, out_ref):
    g = g_ref[...]
    w = w_ref[...]
    m = m_ref[...]
    acc = jnp.broadcast_to(b_ref[...], (BLK, HID))
    for t in range(9):
        acc = acc + jnp.dot(g[:, t, :] * m[:, t:t + 1], w[t],
                            preferred_element_type=jnp.float32)
    out_ref[...] = acc


def _run_k4(g1, W1p, b1row, mask):
    return pl.pallas_call(
        _k4_body,
        grid=(P // BLK,),
        in_specs=[
            pl.BlockSpec((BLK, 9, 256), lambda i: (i, 0, 0)),
            pl.BlockSpec((9, 256, 128), lambda i: (0, 0, 0)),
            pl.BlockSpec((1, 128), lambda i: (0, 0)),
            pl.BlockSpec((BLK, 16), lambda i: (i, 0)),
        ],
        out_specs=pl.BlockSpec((BLK, HID), lambda i: (i, 0)),
        out_shape=jax.ShapeDtypeStruct((P, HID), jnp.float32),
    )(g1, W1p, b1row, mask)


def _k5_body(g_ref, w_ref, xz_ref, m_ref, q_ref, tab_ref):
    g = g_ref[...]
    w = w_ref[...]
    m = m_ref[...]
    acc = jnp.zeros((BLK, 3 * HID), jnp.float32)
    for t in range(9):
        acc = acc + jnp.dot(g[:, t, :] * m[:, t:t + 1], w[t],
                            preferred_element_type=jnp.float32)
    q_ref[...] = acc[:, 0:HID]
    xz = xz_ref[...]
    xyp = jnp.concatenate(
        [xz[:, 0:2], xz[:, 3:4], jnp.zeros((BLK, 125), jnp.float32)], axis=1)
    tab_ref[...] = jnp.concatenate(
        [acc[:, HID:2 * HID], acc[:, 2 * HID:3 * HID], xyp], axis=1)


def _run_k5(g2, W2g, xyzp4, mask):
    return pl.pallas_call(
        _k5_body,
        grid=(P // BLK,),
        in_specs=[
            pl.BlockSpec((BLK, 9, 128), lambda i: (i, 0, 0)),
            pl.BlockSpec((9, 128, 384), lambda i: (0, 0, 0)),
            pl.BlockSpec((BLK, 4), lambda i: (i, 0)),
            pl.BlockSpec((BLK, 16), lambda i: (i, 0)),
        ],
        out_specs=[
            pl.BlockSpec((BLK, HID), lambda i: (i, 0)),
            pl.BlockSpec((BLK, 384), lambda i: (i, 0)),
        ],
        out_shape=[
            jax.ShapeDtypeStruct((P, HID), jnp.float32),
            jax.ShapeDtypeStruct((P, 384), jnp.float32),
        ],
    )(g2, W2g, xyzp4, mask)


def _k6_body(q_ref, g_ref, xz_ref, f_ref,
             pew1_ref, peb1_ref, pew2_ref, peb2_ref,
             aw1_ref, ab1_ref, aw2_ref, ab2_ref,
             agw_ref, agb_ref, mw1_ref, mb1_ref, mw2_ref, mb2_ref,
             out_ref):
    R = QBLK * KNN
    g = g_ref[...]
    xz = xz_ref[...]
    xyp_q = jnp.concatenate(
        [xz[:, 0:2], xz[:, 3:4], jnp.zeros((QBLK, 29), jnp.float32)], axis=1)
    xq = jnp.reshape(
        jnp.broadcast_to(jnp.reshape(xyp_q, (QBLK, 1, 32)), (QBLK, KNN, 32)),
        (R, 32))
    rel = xq - g[:, 256:288]
    pe = jnp.maximum(
        jnp.dot(rel, pew1_ref[...], preferred_element_type=jnp.float32)
        + peb1_ref[...], 0.0)
    pe = jnp.dot(pe, pew2_ref[...],
                 preferred_element_type=jnp.float32) + peb2_ref[...]
    q = q_ref[...]
    qe = jnp.reshape(
        jnp.broadcast_to(jnp.reshape(q, (QBLK, 1, HID)), (QBLK, KNN, HID)),
        (R, HID))
    att = qe - g[:, 0:HID] + pe
    att = jnp.maximum(
        jnp.dot(att, aw1_ref[...], preferred_element_type=jnp.float32)
        + ab1_ref[...], 0.0)
    att = jnp.dot(att, aw2_ref[...],
                  preferred_element_type=jnp.float32) + ab2_ref[...]
    att = att / math.sqrt(HID)
    a3 = jnp.reshape(att, (QBLK, KNN, HID))
    mx = jnp.max(a3, axis=1, keepdims=True)
    e = jnp.exp(a3 - mx)
    sm = e / jnp.sum(e, axis=1, keepdims=True)
    v3 = jnp.reshape(g[:, HID:2 * HID] + pe, (QBLK, KNN, HID))
    res = jnp.sum(sm * v3, axis=1)
    f = f_ref[...]
    res = jnp.dot(res, agw_ref[...],
                  preferred_element_type=jnp.float32) + agb_ref[...] + f
    h = jnp.dot(res, mw1_ref[...],
                preferred_element_type=jnp.float32) + mb1_ref[...]
    h = 0.5 * h * (1.0 + lax.erf(h / math.sqrt(2.0)))
    out_ref[...] = jnp.dot(h, mw2_ref[...],
                           preferred_element_type=jnp.float32) + mb2_ref[...] + f


def _run_k6(q, ga, xyzp4, feat2, weights):
    full = lambda r, c: pl.BlockSpec((r, c), lambda i: (0, 0))
    return pl.pallas_call(
        _k6_body,
        grid=(P // QBLK,),
        in_specs=[
            pl.BlockSpec((QBLK, HID), lambda i: (i, 0)),
            pl.BlockSpec((QBLK * KNN, 384), lambda i: (i, 0)),
            pl.BlockSpec((QBLK, 4), lambda i: (i, 0)),
            pl.BlockSpec((QBLK, 128), lambda i: (i, 0)),
            full(32, 128), full(1, 128), full(128, 128), full(1, 128),
            full(128, 128), full(1, 128), full(128, 128), full(1, 128),
            full(128, 128), full(1, 128), full(128, 128), full(1, 128),
            full(128, 128), full(1, 128),
        ],
        out_specs=pl.BlockSpec((QBLK, FEAT), lambda i: (i, 0)),
        out_shape=jax.ShapeDtypeStruct((P, FEAT), jnp.float32),
    )(q, ga, xyzp4, feat2, *weights)


def kernel(xyzp, features, gamma, beta, W1, b1, W2, Pe_W1, Pe_b1, Pe_W2,
           Pe_b2, A_W1, A_b1, A_W2, A_b2, Ag_W, Ag_b, M_W1, M_b1, M_W2, M_b2):
    xyzp4 = xyzp.reshape(P, 4)
    feat2 = features.reshape(P, FEAT)
    g_row = jnp.pad(gamma, (0, 126)).reshape(1, 256)
    b_row = jnp.pad(beta, (0, 126)).reshape(1, 256)
    xyT = jnp.pad(jnp.transpose(xyzp[..., 0:2], (0, 2, 1)),
                  ((0, 0), (0, 6), (0, 0)))
    xyzpT = xyzp4.T
    W1p = jnp.pad(W1.reshape(9, FEAT + 2, HID), ((0, 0), (0, 126), (0, 0)))
    perm = jnp.concatenate(
        [3 * jnp.arange(HID, dtype=jnp.int32) + c for c in range(3)])
    W2g = W2.reshape(9, HID, 3 * HID)[:, :, perm]
    b1row = b1.reshape(1, HID)
    row = lambda x: x.reshape(1, 128)
    pew1p = jnp.pad(Pe_W1, ((0, 29), (0, 0)))

    sparse, idx, nidx, mask = _run_k123(
        xyzp4, feat2, g_row, b_row, xyT, xyzpT)
    conv_idx = nidx[:, :9].reshape(-1)
    att_idx = idx.reshape(-1)

    g1 = _sc_gather(sparse, conv_idx).reshape(P, 9, 256)
    h1 = _run_k4(g1, W1p, b1row, mask)
    g2 = _sc_gather(h1, conv_idx).reshape(P, 9, HID)
    q, att_table = _run_k5(g2, W2g, xyzp4, mask)
    ga = _sc_gather(att_table, att_idx)

    weights = (pew1p, row(Pe_b1), Pe_W2, row(Pe_b2),
               A_W1, row(A_b1), A_W2, row(A_b2),
               Ag_W, row(Ag_b), M_W1, row(M_b1), M_W2, row(M_b2))
    out = _run_k6(q, ga, xyzp4, feat2, weights)
    return out.reshape(B, N, FEAT)
